# baseline (speedup 1.0000x reference)
import functools

import jax
import jax.numpy as jnp
from jax import lax
from jax.experimental import pallas as pl
from jax.experimental.pallas import tpu as pltpu
from jax.experimental.pallas import tpu_sc as plsc

N_NODES = 10000
N_PAD = 10240
D = 128
DJ = D // 16
N_EDGES = 320000
CW = 16
NC, NS = 2, 16
NW = NC * NS
E_PER_W = N_EDGES // NW
BE = 80
NB = E_PER_W // BE
CB = 25
CH = CB * BE
ROWS_PER_TILE = N_PAD // NS
NBUF = 3

_MAIN = (NB - 2) // NBUF * NBUF
_TAIL = list(range(_MAIN, NB))


def _hop_body(with_counts, ego_hbm, idx3_hbm, rel_hbm, *refs):
    if with_counts:
        (acc_out, cnt_out, acc_sh, cnt_sh, it_v, rel_t, ones_v, zc_v,
         hs, rows, gsems, ssems, hsems, osem) = (
            refs[0], refs[1], refs[2], refs[3], refs[4], refs[5], refs[6],
            refs[7], refs[8:11], refs[11:14], refs[14:17], refs[17:20],
            refs[20:23], refs[23])
    else:
        (acc_out, acc_sh, it_v, rel_t, hs, rows, gsems, ssems, hsems) = (
            refs[0], refs[1], refs[2], refs[3], refs[4:7], refs[7:10],
            refs[10:13], refs[13:16], refs[16:19])

    cid = lax.axis_index("c")
    sid = lax.axis_index("s")
    wid = sid * NC + cid
    base_e = wid * E_PER_W

    zero16 = jnp.zeros((16,), jnp.float32)

    pltpu.sync_copy(rel_hbm, rel_t)

    @plsc.parallel_loop(0, BE * DJ, unroll=8)
    def _(i):
        rows[0][i // DJ, pl.ds((i % DJ) * 16, 16)] = zero16

    @pl.loop(0, ROWS_PER_TILE // BE)
    def _(j):
        pltpu.sync_copy(rows[0], acc_sh.at[pl.ds(sid * ROWS_PER_TILE + j * BE, BE)])

    if with_counts:
        one16 = jnp.ones((16,), jnp.float32)

        @pl.loop(0, BE * CW // 16)
        def _(i):
            ones_v[i // (CW // 16), pl.ds((i % (CW // 16)) * 16, 16)] = one16

        @pl.loop(0, 64 * CW // 16)
        def _(i):
            zc_v[i // (CW // 16), pl.ds((i % (CW // 16)) * 16, 16)] = zero16

        @pl.loop(0, ROWS_PER_TILE // 64)
        def _(j):
            pltpu.sync_copy(
                zc_v, cnt_sh.at[pl.ds(sid * ROWS_PER_TILE + j * 64, 64)])

    plsc.subcore_barrier()

    def load_chunk(b):
        hoff = base_e + (b // CB) * CH
        pltpu.sync_copy(idx3_hbm.at[:, pl.ds(hoff, CH)], it_v)

    def gather_desc(b, k):
        coff = (b % CB) * BE
        return pltpu.make_async_copy(
            ego_hbm.at[it_v.at[0, pl.ds(coff, BE)]], rows[k], gsems[k])

    def issue_gather(b, k):
        coff = (b % CB) * BE
        pltpu.async_copy(
            ego_hbm.at[it_v.at[0, pl.ds(coff, BE)]], rows[k], gsems[k])
        pltpu.async_copy(
            idx3_hbm.at[1, pl.ds(base_e + b * BE, BE)], hs[k], hsems[k])

    def head_desc(b, k):
        return pltpu.make_async_copy(
            idx3_hbm.at[1, pl.ds(base_e + b * BE, BE)], hs[k], hsems[k])

    def scat_desc(k):
        return pltpu.make_async_copy(rows[k], acc_sh.at[hs[k]], ssems[k])

    @pl.loop(0, NB)
    def _(b):
        @pl.when(b % CB == 0)
        def _():
            load_chunk(b)

        issue_gather(b, 0)
        gather_desc(b, 0).wait()
        head_desc(b, 0).wait()

        coff = (b % CB) * BE

        iota16 = lax.iota(jnp.int32, 16)
        for g in range(BE // 16):
            tvec = it_v[2, pl.ds(coff + g * 16, 16)]
            e_idx = iota16 + (g * 16)

            @plsc.parallel_loop(0, D, unroll=4)
            def _(c):
                cvec = jnp.zeros((16,), jnp.int32) + c
                ve = plsc.load_gather(rows[0], [e_idx, cvec])
                vr = plsc.load_gather(rel_t, [tvec, cvec])
                plsc.store_scatter(rows[0], [e_idx, cvec], ve * vr)

        pltpu.sync_copy(rows[0], acc_sh.at[hs[0]], add=True)
        if with_counts:
            pltpu.sync_copy(ones_v, cnt_sh.at[hs[0]], add=True)

    plsc.subcore_barrier()
    r0 = sid * ROWS_PER_TILE
    pltpu.sync_copy(acc_sh.at[pl.ds(r0, ROWS_PER_TILE)],
                    acc_out.at[cid, pl.ds(r0, ROWS_PER_TILE)])
    if with_counts:
        pltpu.sync_copy(cnt_sh.at[pl.ds(r0, ROWS_PER_TILE)],
                        cnt_out.at[cid, pl.ds(r0, ROWS_PER_TILE)])


_MESH = plsc.VectorSubcoreMesh(core_axis_name="c", subcore_axis_name="s")

_agg_sc1 = pl.kernel(
    functools.partial(_hop_body, True),
    out_type=[
        jax.ShapeDtypeStruct((NC, N_PAD, D), jnp.float32),
        jax.ShapeDtypeStruct((NC, N_PAD, CW), jnp.float32),
    ],
    mesh=_MESH,
    compiler_params=pltpu.CompilerParams(use_tc_tiling_on_sc=False, needs_layout_passes=False),
    scratch_types=[
        pltpu.VMEM_SHARED((N_PAD, D), jnp.float32),
        pltpu.VMEM_SHARED((N_PAD, CW), jnp.float32),
        pltpu.VMEM((3, CH), jnp.int32),
        pltpu.VMEM((16, D), jnp.float32),
        pltpu.VMEM((BE, CW), jnp.float32),
        pltpu.VMEM((64, CW), jnp.float32),
    ] + [pltpu.VMEM((BE,), jnp.int32)] * NBUF
      + [pltpu.VMEM((BE, D), jnp.float32)] * NBUF
      + [pltpu.SemaphoreType.DMA] * (3 * NBUF + 1),
)

_agg_sc2 = pl.kernel(
    functools.partial(_hop_body, False),
    out_type=[
        jax.ShapeDtypeStruct((NC, N_PAD, D), jnp.float32),
    ],
    mesh=_MESH,
    compiler_params=pltpu.CompilerParams(use_tc_tiling_on_sc=False, needs_layout_passes=False),
    scratch_types=[
        pltpu.VMEM_SHARED((N_PAD, D), jnp.float32),
        pltpu.VMEM((3, CH), jnp.int32),
        pltpu.VMEM((16, D), jnp.float32),
    ] + [pltpu.VMEM((BE,), jnp.int32)] * NBUF
      + [pltpu.VMEM((BE, D), jnp.float32)] * NBUF
      + [pltpu.SemaphoreType.DMA] * (3 * NBUF),
)


ROW_BLK = 1024


def _norm_body(acc_ref, cnt_ref, res_ref, ego_out_ref, res_out_ref):
    a = acc_ref[0] + acc_ref[1]
    c = cnt_ref[0, :, 0:1] + cnt_ref[1, :, 0:1]
    mean = a / jnp.maximum(c, 1.0)
    n = jnp.sqrt(jnp.sum(mean * mean, axis=1, keepdims=True))
    ego = mean / jnp.maximum(n, 1e-12)
    ego_out_ref[...] = ego
    res_out_ref[...] = res_ref[...] + ego


def _norm_tc(acc, cnt, res):
    grid = (N_PAD // ROW_BLK,)
    return pl.pallas_call(
        _norm_body,
        grid=grid,
        in_specs=[
            pl.BlockSpec((NC, ROW_BLK, D), lambda i: (0, i, 0)),
            pl.BlockSpec((NC, ROW_BLK, CW), lambda i: (0, i, 0)),
            pl.BlockSpec((ROW_BLK, D), lambda i: (i, 0)),
        ],
        out_specs=[
            pl.BlockSpec((ROW_BLK, D), lambda i: (i, 0)),
            pl.BlockSpec((ROW_BLK, D), lambda i: (i, 0)),
        ],
        out_shape=[
            jax.ShapeDtypeStruct((N_PAD, D), jnp.float32),
            jax.ShapeDtypeStruct((N_PAD, D), jnp.float32),
        ],
    )(acc, cnt, res)


@jax.jit
def kernel(ego_embed, edge_index, edge_type, relation_embed, dropout):
    head = edge_index[0].astype(jnp.int32)
    tail = edge_index[1].astype(jnp.int32)
    typ = edge_type.astype(jnp.int32)
    idx3 = jnp.stack([tail, head, typ])
    ego = jnp.pad(ego_embed, ((0, N_PAD - N_NODES), (0, 0)))
    res = ego

    acc, cnt = _agg_sc1(ego, idx3, relation_embed)
    ego, res = _norm_tc(acc, cnt, res)
    (acc,) = _agg_sc2(ego, idx3, relation_embed)
    ego, res = _norm_tc(acc, cnt, res)
    return res[:N_NODES]


# speedup vs baseline: 1.1734x; 1.1734x over previous
import functools

import jax
import jax.numpy as jnp
from jax import lax
from jax.experimental import pallas as pl
from jax.experimental.pallas import tpu as pltpu
from jax.experimental.pallas import tpu_sc as plsc

N_NODES = 10000
N_PAD = 10240
D = 128
DJ = D // 16
N_EDGES = 320000
CW = 16
NC, NS = 2, 16
NW = NC * NS
E_PER_W = N_EDGES // NW
BE = 80
NB = E_PER_W // BE
CB = 15
CH = CB * BE
ROWS_PER_TILE = N_PAD // NS
NBUF = 3

_MAIN = (NB - 2) // NBUF * NBUF
_TAIL = list(range(_MAIN, NB))


def _hop_body(with_counts, ego_hbm, idx3_hbm, rel_hbm, *refs):
    if with_counts:
        (acc_out, cnt_out, acc_sh, cnt_sh, it_v, rel_t, ones_v, zc_v,
         hs, rows, gsems, ssems, hsems, osem) = (
            refs[0], refs[1], refs[2], refs[3], refs[4], refs[5], refs[6],
            refs[7], refs[8:11], refs[11:14], refs[14:17], refs[17:20],
            refs[20:23], refs[23])
    else:
        (acc_out, acc_sh, it_v, rel_t, hs, rows, gsems, ssems, hsems) = (
            refs[0], refs[1], refs[2], refs[3], refs[4:7], refs[7:10],
            refs[10:13], refs[13:16], refs[16:19])

    cid = lax.axis_index("c")
    sid = lax.axis_index("s")
    wid = sid * NC + cid
    base_e = wid * E_PER_W

    zero16 = jnp.zeros((16,), jnp.float32)

    pltpu.sync_copy(rel_hbm, rel_t)

    @plsc.parallel_loop(0, BE * DJ, unroll=8)
    def _(i):
        rows[0][i // DJ, pl.ds((i % DJ) * 16, 16)] = zero16

    @pl.loop(0, ROWS_PER_TILE // BE)
    def _(j):
        pltpu.sync_copy(rows[0], acc_sh.at[pl.ds(sid * ROWS_PER_TILE + j * BE, BE)])

    if with_counts:
        one16 = jnp.ones((16,), jnp.float32)

        @pl.loop(0, BE * CW // 16)
        def _(i):
            ones_v[i // (CW // 16), pl.ds((i % (CW // 16)) * 16, 16)] = one16

        @pl.loop(0, 32 * CW // 16)
        def _(i):
            zc_v[i // (CW // 16), pl.ds((i % (CW // 16)) * 16, 16)] = zero16

        @pl.loop(0, ROWS_PER_TILE // 32)
        def _(j):
            pltpu.sync_copy(
                zc_v, cnt_sh.at[pl.ds(sid * ROWS_PER_TILE + j * 32, 32)])

    plsc.subcore_barrier()

    def load_chunk(b):
        hoff = base_e + (b // CB) * CH
        pltpu.sync_copy(idx3_hbm.at[:, pl.ds(hoff, CH)], it_v)

    def gather_desc(b, k):
        coff = (b % CB) * BE
        return pltpu.make_async_copy(
            ego_hbm.at[it_v.at[0, pl.ds(coff, BE)]], rows[k], gsems[k])

    def issue_gather(b, k):
        coff = (b % CB) * BE
        pltpu.async_copy(
            ego_hbm.at[it_v.at[0, pl.ds(coff, BE)]], rows[k], gsems[k])
        pltpu.async_copy(
            idx3_hbm.at[1, pl.ds(base_e + b * BE, BE)], hs[k], hsems[k])

    def head_desc(b, k):
        return pltpu.make_async_copy(
            idx3_hbm.at[1, pl.ds(base_e + b * BE, BE)], hs[k], hsems[k])

    def scat_desc(k):
        return pltpu.make_async_copy(rows[k], acc_sh.at[hs[k]], ssems[k])

    def step(b, k, issue_next, guard_scat_wait, last):
        nk = (k + 1) % NBUF
        pk = (k - 1) % NBUF
        coff = (b % CB) * BE
        tvecs = [it_v[2, pl.ds(coff + g * 16, 16)] for g in range(BE // 16)]
        gather_desc(b, k).wait()
        if issue_next:
            @pl.when((b + 1) % CB == 0)
            def _():
                load_chunk(b + 1)

            if guard_scat_wait:
                @pl.when(b + 1 >= NBUF)
                def _():
                    scat_desc(nk).wait()
            else:
                scat_desc(nk).wait()
            issue_gather(b + 1, nk)

        iota16 = lax.iota(jnp.int32, 16)
        for g in range(BE // 16):
            tvec = tvecs[g]
            e_idx = iota16 + (g * 16)

            @plsc.parallel_loop(0, D, unroll=4)
            def _(c):
                cvec = jnp.zeros((16,), jnp.int32) + c
                ve = plsc.load_gather(rows[k], [e_idx, cvec])
                vr = plsc.load_gather(rel_t, [tvec, cvec])
                plsc.store_scatter(rows[k], [e_idx, cvec], ve * vr)

        head_desc(b, k).wait()
        pltpu.async_copy(rows[k], acc_sh.at[hs[k]], ssems[k], add=True)
        if with_counts:
            @pl.when(b >= 1)
            def _():
                pltpu.make_async_copy(ones_v, cnt_sh.at[hs[pk]], osem).wait()
            pltpu.async_copy(ones_v, cnt_sh.at[hs[k]], osem, add=True)

    load_chunk(0)
    issue_gather(0, 0)

    @pl.loop(0, _MAIN // NBUF)
    def _(g):
        b0 = g * NBUF
        step(b0, 0, True, True, False)
        step(b0 + 1, 1, True, True, False)
        step(b0 + 2, 2, True, True, False)

    for i, b in enumerate(_TAIL):
        step(jnp.int32(b), b % NBUF, b + 1 < NB, False, b + 1 >= NB)

    for k in range(NBUF):
        scat_desc(k).wait()
    if with_counts:
        pltpu.make_async_copy(ones_v, cnt_sh.at[hs[_TAIL[-1] % NBUF]],
                              osem).wait()

    plsc.subcore_barrier()
    r0 = sid * ROWS_PER_TILE
    pltpu.sync_copy(acc_sh.at[pl.ds(r0, ROWS_PER_TILE)],
                    acc_out.at[cid, pl.ds(r0, ROWS_PER_TILE)])
    if with_counts:
        pltpu.sync_copy(cnt_sh.at[pl.ds(r0, ROWS_PER_TILE)],
                        cnt_out.at[cid, pl.ds(r0, ROWS_PER_TILE)])


_MESH = plsc.VectorSubcoreMesh(core_axis_name="c", subcore_axis_name="s")

_agg_sc1 = pl.kernel(
    functools.partial(_hop_body, True),
    out_type=[
        jax.ShapeDtypeStruct((NC, N_PAD, D), jnp.float32),
        jax.ShapeDtypeStruct((NC, N_PAD, CW), jnp.float32),
    ],
    mesh=_MESH,
    compiler_params=pltpu.CompilerParams(use_tc_tiling_on_sc=False, needs_layout_passes=False),
    scratch_types=[
        pltpu.VMEM_SHARED((N_PAD, D), jnp.float32),
        pltpu.VMEM_SHARED((N_PAD, CW), jnp.float32),
        pltpu.VMEM((3, CH), jnp.int32),
        pltpu.VMEM((16, D), jnp.float32),
        pltpu.VMEM((BE, CW), jnp.float32),
        pltpu.VMEM((32, CW), jnp.float32),
    ] + [pltpu.VMEM((BE,), jnp.int32)] * NBUF
      + [pltpu.VMEM((BE, D), jnp.float32)] * NBUF
      + [pltpu.SemaphoreType.DMA] * (3 * NBUF + 1),
)

_agg_sc2 = pl.kernel(
    functools.partial(_hop_body, False),
    out_type=[
        jax.ShapeDtypeStruct((NC, N_PAD, D), jnp.float32),
    ],
    mesh=_MESH,
    compiler_params=pltpu.CompilerParams(use_tc_tiling_on_sc=False, needs_layout_passes=False),
    scratch_types=[
        pltpu.VMEM_SHARED((N_PAD, D), jnp.float32),
        pltpu.VMEM((3, CH), jnp.int32),
        pltpu.VMEM((16, D), jnp.float32),
    ] + [pltpu.VMEM((BE,), jnp.int32)] * NBUF
      + [pltpu.VMEM((BE, D), jnp.float32)] * NBUF
      + [pltpu.SemaphoreType.DMA] * (3 * NBUF),
)


ROW_BLK = 1024


def _norm_body(acc_ref, cnt_ref, res_ref, ego_out_ref, res_out_ref):
    a = acc_ref[0] + acc_ref[1]
    c = cnt_ref[0, :, 0:1] + cnt_ref[1, :, 0:1]
    mean = a / jnp.maximum(c, 1.0)
    n = jnp.sqrt(jnp.sum(mean * mean, axis=1, keepdims=True))
    ego = mean / jnp.maximum(n, 1e-12)
    ego_out_ref[...] = ego
    res_out_ref[...] = res_ref[...] + ego


def _norm_tc(acc, cnt, res):
    grid = (N_PAD // ROW_BLK,)
    return pl.pallas_call(
        _norm_body,
        grid=grid,
        in_specs=[
            pl.BlockSpec((NC, ROW_BLK, D), lambda i: (0, i, 0)),
            pl.BlockSpec((NC, ROW_BLK, CW), lambda i: (0, i, 0)),
            pl.BlockSpec((ROW_BLK, D), lambda i: (i, 0)),
        ],
        out_specs=[
            pl.BlockSpec((ROW_BLK, D), lambda i: (i, 0)),
            pl.BlockSpec((ROW_BLK, D), lambda i: (i, 0)),
        ],
        out_shape=[
            jax.ShapeDtypeStruct((N_PAD, D), jnp.float32),
            jax.ShapeDtypeStruct((N_PAD, D), jnp.float32),
        ],
    )(acc, cnt, res)


@jax.jit
def kernel(ego_embed, edge_index, edge_type, relation_embed, dropout):
    head = edge_index[0].astype(jnp.int32)
    tail = edge_index[1].astype(jnp.int32)
    typ = edge_type.astype(jnp.int32)
    idx3 = jnp.pad(jnp.stack([tail, head, typ]), ((0, 0), (0, CH)))
    ego = jnp.pad(ego_embed, ((0, N_PAD - N_NODES), (0, 0)))
    res = ego

    acc, cnt = _agg_sc1(ego, idx3, relation_embed)
    ego, res = _norm_tc(acc, cnt, res)
    (acc,) = _agg_sc2(ego, idx3, relation_embed)
    ego, res = _norm_tc(acc, cnt, res)
    return res[:N_NODES]


# speedup vs baseline: 1.4251x; 1.2145x over previous
import functools

import jax
import jax.numpy as jnp
from jax import lax
from jax.experimental import pallas as pl
from jax.experimental.pallas import tpu as pltpu
from jax.experimental.pallas import tpu_sc as plsc

N_NODES = 10000
N_PAD = 10240
D = 128
N_EDGES = 320000
CW = 16
NC, NS = 2, 16
NW = NC * NS
E_PER_W = N_EDGES // NW
BE = 80
NB = E_PER_W // BE
ROWS_PER_TILE = N_PAD // NS
ZR = 64


def _agg_body(ego_hbm, tail_hbm, head_hbm, type_hbm, rel_hbm,
              acc_out, cnt_out,
              acc_sh, cnt_sh, tail_v, head_v, type_v, rows_v, rel_v,
              zb_v, zc_v, ones_v, sem1, sem2):
    cid = lax.axis_index("c")
    sid = lax.axis_index("s")
    wid = sid * NC + cid

    zero16 = jnp.zeros((16,), jnp.float32)
    one16 = jnp.ones((16,), jnp.float32)

    @pl.loop(0, ZR * (D // 16))
    def _(i):
        zb_v[i // (D // 16), pl.ds((i % (D // 16)) * 16, 16)] = zero16

    @pl.loop(0, ZR)
    def _(r):
        zc_v[r, :] = zero16

    @pl.loop(0, BE)
    def _(r):
        ones_v[r, :] = one16

    @pl.loop(0, ROWS_PER_TILE // ZR)
    def _(j):
        pltpu.sync_copy(
            zb_v, acc_sh.at[pl.ds(sid * ROWS_PER_TILE + j * ZR, ZR)])
        pltpu.sync_copy(
            zc_v, cnt_sh.at[pl.ds(sid * ROWS_PER_TILE + j * ZR, ZR)])
    plsc.subcore_barrier()

    base_e = wid * E_PER_W

    @pl.loop(0, NB)
    def _(b):
        off = base_e + b * BE
        pltpu.sync_copy(tail_hbm.at[pl.ds(off, BE)], tail_v)
        pltpu.sync_copy(head_hbm.at[pl.ds(off, BE)], head_v)
        pltpu.sync_copy(type_hbm.at[pl.ds(off, BE)], type_v)
        c1 = pltpu.async_copy(ego_hbm.at[tail_v], rows_v, sem1)
        c2 = pltpu.async_copy(rel_hbm.at[type_v], rel_v, sem2)
        c1.wait()
        c2.wait()

        @plsc.parallel_loop(0, BE * (D // 16), unroll=8)
        def _(i):
            e = i // (D // 16)
            jj = (i % (D // 16)) * 16
            rows_v[e, pl.ds(jj, 16)] = (
                rows_v[e, pl.ds(jj, 16)] * rel_v[e, pl.ds(jj, 16)])

        pltpu.sync_copy(rows_v, acc_sh.at[head_v], add=True)
        pltpu.sync_copy(ones_v, cnt_sh.at[head_v], add=True)

    plsc.subcore_barrier()
    r0 = sid * ROWS_PER_TILE
    pltpu.sync_copy(acc_sh.at[pl.ds(r0, ROWS_PER_TILE)],
                    acc_out.at[cid, pl.ds(r0, ROWS_PER_TILE)])
    pltpu.sync_copy(cnt_sh.at[pl.ds(r0, ROWS_PER_TILE)],
                    cnt_out.at[cid, pl.ds(r0, ROWS_PER_TILE)])


_agg_sc = pl.kernel(
    _agg_body,
    out_type=[
        jax.ShapeDtypeStruct((NC, N_PAD, D), jnp.float32),
        jax.ShapeDtypeStruct((NC, N_PAD, CW), jnp.float32),
    ],
    mesh=plsc.VectorSubcoreMesh(core_axis_name="c", subcore_axis_name="s"),
    compiler_params=pltpu.CompilerParams(use_tc_tiling_on_sc=False),
    scratch_types=[
        pltpu.VMEM_SHARED((N_PAD, D), jnp.float32),
        pltpu.VMEM_SHARED((N_PAD, CW), jnp.float32),
        pltpu.VMEM((BE,), jnp.int32),
        pltpu.VMEM((BE,), jnp.int32),
        pltpu.VMEM((BE,), jnp.int32),
        pltpu.VMEM((BE, D), jnp.float32),
        pltpu.VMEM((BE, D), jnp.float32),
        pltpu.VMEM((ZR, D), jnp.float32),
        pltpu.VMEM((ZR, CW), jnp.float32),
        pltpu.VMEM((BE, CW), jnp.float32),
        pltpu.SemaphoreType.DMA,
        pltpu.SemaphoreType.DMA,
    ],
)


ROW_BLK = 1024


def _norm_body(acc_ref, cnt_ref, res_ref, ego_out_ref, res_out_ref):
    a = acc_ref[0] + acc_ref[1]
    c = cnt_ref[0, :, 0:1] + cnt_ref[1, :, 0:1]
    mean = a / jnp.maximum(c, 1.0)
    n = jnp.sqrt(jnp.sum(mean * mean, axis=1, keepdims=True))
    ego = mean / jnp.maximum(n, 1e-12)
    ego_out_ref[...] = ego
    res_out_ref[...] = res_ref[...] + ego


def _norm_tc(acc, cnt, res):
    grid = (N_PAD // ROW_BLK,)
    return pl.pallas_call(
        _norm_body,
        grid=grid,
        in_specs=[
            pl.BlockSpec((NC, ROW_BLK, D), lambda i: (0, i, 0)),
            pl.BlockSpec((NC, ROW_BLK, CW), lambda i: (0, i, 0)),
            pl.BlockSpec((ROW_BLK, D), lambda i: (i, 0)),
        ],
        out_specs=[
            pl.BlockSpec((ROW_BLK, D), lambda i: (i, 0)),
            pl.BlockSpec((ROW_BLK, D), lambda i: (i, 0)),
        ],
        out_shape=[
            jax.ShapeDtypeStruct((N_PAD, D), jnp.float32),
            jax.ShapeDtypeStruct((N_PAD, D), jnp.float32),
        ],
    )(acc, cnt, res)


@jax.jit
def kernel(ego_embed, edge_index, edge_type, relation_embed, dropout):
    head = edge_index[0].astype(jnp.int32)
    tail = edge_index[1].astype(jnp.int32)
    typ = edge_type.astype(jnp.int32)
    ego = jnp.pad(ego_embed, ((0, N_PAD - N_NODES), (0, 0)))
    res = ego
    for _ in range(2):
        acc, cnt = _agg_sc(ego, tail, head, typ, relation_embed)
        ego, res = _norm_tc(acc, cnt, res)
    return res[:N_NODES]
